# baseline (speedup 1.0000x reference)
import jax
import jax.numpy as jnp
from jax.experimental import pallas as pl

N = 10000
E = 320000
H = 128
NR = 20
BE = 2000


def _edge_mlp_body(rbf_ref, ssrc_ref, w1f_ref, b1f_ref, w2f_ref, b2f_ref,
                   w1s_ref, b1s_ref, w2s_ref, b2s_ref, msg_ref):
    h1 = jax.nn.silu(
        jnp.dot(rbf_ref[...], w1f_ref[...], preferred_element_type=jnp.float32)
        + b1f_ref[...])
    filt = jnp.dot(h1, w2f_ref[...], preferred_element_type=jnp.float32) + b2f_ref[...]
    h2 = jax.nn.silu(
        jnp.dot(ssrc_ref[...], w1s_ref[...], preferred_element_type=jnp.float32)
        + b1s_ref[...])
    scal = jnp.dot(h2, w2s_ref[...], preferred_element_type=jnp.float32) + b2s_ref[...]
    msg_ref[...] = filt * scal


def _edge_mlp(rbf, s_src, W1f, b1f, W2f, b2f, W1s, b1s, W2s, b2s):
    grid = (E // BE,)
    return pl.pallas_call(
        _edge_mlp_body,
        grid=grid,
        in_specs=[
            pl.BlockSpec((BE, NR), lambda i: (i, 0)),
            pl.BlockSpec((BE, H), lambda i: (i, 0)),
            pl.BlockSpec((NR, H), lambda i: (0, 0)),
            pl.BlockSpec((1, H), lambda i: (0, 0)),
            pl.BlockSpec((H, 3 * H), lambda i: (0, 0)),
            pl.BlockSpec((1, 3 * H), lambda i: (0, 0)),
            pl.BlockSpec((H, H), lambda i: (0, 0)),
            pl.BlockSpec((1, H), lambda i: (0, 0)),
            pl.BlockSpec((H, 3 * H), lambda i: (0, 0)),
            pl.BlockSpec((1, 3 * H), lambda i: (0, 0)),
        ],
        out_specs=pl.BlockSpec((BE, 3 * H), lambda i: (i, 0)),
        out_shape=jax.ShapeDtypeStruct((E, 3 * H), jnp.float32),
    )(rbf, s_src, W1f, b1f.reshape(1, H), W2f, b2f.reshape(1, 3 * H),
      W1s, b1s.reshape(1, H), W2s, b2s.reshape(1, 3 * H))


def kernel(s, v, edge_index, rbf, unit, W1f, b1f, W2f, b2f, W1s, b1s, W2s, b2s):
    src = edge_index[0]
    dst = edge_index[1]
    s_src = jnp.take(s, src, axis=0)
    msg = _edge_mlp(rbf, s_src, W1f, b1f, W2f, b2f, W1s, b1s, W2s, b2s)
    ds, dv_vector, dv_radial = jnp.split(msg, 3, axis=-1)
    v_src = jnp.take(v, src, axis=0)
    dv = dv_vector[:, None, :] * v_src + dv_radial[:, None, :] * unit[:, :, None]
    s_out = s.at[dst].add(ds)
    v_out = v.at[dst].add(dv)
    return (s_out, v_out)


# speedup vs baseline: 11.2728x; 11.2728x over previous
import functools

import jax
import jax.numpy as jnp
from jax import lax
from jax.experimental import pallas as pl
from jax.experimental.pallas import tpu as pltpu
from jax.experimental.pallas import tpu_sc as plsc

N = 10000
E = 320000
H = 128
Q = 32
NR = 20
BE = 2000

NC = 2
NS = 16
CHUNK = 80


def _edge_mlp_body(rbf_ref, ssrc_ref, w1f_ref, b1f_ref, w2f_ref, b2f_ref,
                   w1s_ref, b1s_ref, w2s_ref, b2s_ref, msgs_ref, msgv_ref):
    h1 = jax.nn.silu(
        jnp.dot(rbf_ref[...], w1f_ref[...], preferred_element_type=jnp.float32)
        + b1f_ref[...])
    filt = jnp.dot(h1, w2f_ref[...], preferred_element_type=jnp.float32) + b2f_ref[...]
    h2 = jax.nn.silu(
        jnp.dot(ssrc_ref[...], w1s_ref[...], preferred_element_type=jnp.float32)
        + b1s_ref[...])
    scal = jnp.dot(h2, w2s_ref[...], preferred_element_type=jnp.float32) + b2s_ref[...]
    msg = filt * scal
    ds = msg[:, :H]
    dvv = msg[:, H:2 * H]
    dvr = msg[:, 2 * H:]
    msgs_ref[...] = ds
    for q in range(4):
        msgv_ref[q] = jnp.concatenate(
            [dvv[:, Q * q:Q * (q + 1)], dvr[:, Q * q:Q * (q + 1)]], axis=-1)


def _edge_mlp(rbf, s_src, W1f, b1f, W2f, b2f, W1s, b1s, W2s, b2s):
    return pl.pallas_call(
        _edge_mlp_body,
        grid=(E // BE,),
        in_specs=[
            pl.BlockSpec((BE, NR), lambda i: (i, 0)),
            pl.BlockSpec((BE, H), lambda i: (i, 0)),
            pl.BlockSpec((NR, H), lambda i: (0, 0)),
            pl.BlockSpec((1, H), lambda i: (0, 0)),
            pl.BlockSpec((H, 3 * H), lambda i: (0, 0)),
            pl.BlockSpec((1, 3 * H), lambda i: (0, 0)),
            pl.BlockSpec((H, H), lambda i: (0, 0)),
            pl.BlockSpec((1, H), lambda i: (0, 0)),
            pl.BlockSpec((H, 3 * H), lambda i: (0, 0)),
            pl.BlockSpec((1, 3 * H), lambda i: (0, 0)),
        ],
        out_specs=[
            pl.BlockSpec((BE, H), lambda i: (i, 0)),
            pl.BlockSpec((4, BE, 2 * Q), lambda i: (0, i, 0)),
        ],
        out_shape=[
            jax.ShapeDtypeStruct((E, H), jnp.float32),
            jax.ShapeDtypeStruct((4, E, 2 * Q), jnp.float32),
        ],
    )(rbf, s_src, W1f, b1f.reshape(1, H), W2f, b2f.reshape(1, 3 * H),
      W1s, b1s.reshape(1, H), W2s, b2s.reshape(1, 3 * H))


_VMESH = plsc.VectorSubcoreMesh(core_axis_name="c", subcore_axis_name="s",
                                num_cores=NC, num_subcores=NS)


@functools.partial(
    pl.kernel,
    out_type=jax.ShapeDtypeStruct((E, H), jnp.float32),
    mesh=_VMESH,
    scratch_types=[
        pltpu.VMEM((CHUNK,), jnp.int32),
        pltpu.VMEM((CHUNK, H), jnp.float32),
        pltpu.SemaphoreType.DMA,
    ],
)
def _sc_gather_s(s_hbm, src_hbm, out_hbm, idx_v, rows_v, sem):
    wid = lax.axis_index("s") * NC + lax.axis_index("c")
    per_w = E // (NC * NS)

    @pl.loop(0, per_w // CHUNK)
    def _(j):
        base = pl.multiple_of(wid * per_w + j * CHUNK, CHUNK)
        pltpu.sync_copy(src_hbm.at[pl.ds(base, CHUNK)], idx_v)
        pltpu.async_copy(s_hbm.at[idx_v], rows_v, sem).wait()
        pltpu.sync_copy(rows_v, out_hbm.at[pl.ds(base, CHUNK)])


@functools.partial(
    pl.kernel,
    out_type=jax.ShapeDtypeStruct((NC, N, H), jnp.float32),
    mesh=_VMESH,
    scratch_types=[
        pltpu.VMEM_SHARED((N, H), jnp.float32),
        pltpu.VMEM((CHUNK,), jnp.int32),
        pltpu.VMEM((CHUNK, H), jnp.float32),
    ],
)
def _sc_scatter_s(s0_hbm, msgs_hbm, dst_hbm, out_hbm, acc_sh, idx_v, upd_v):
    c = lax.axis_index("c")
    sid = lax.axis_index("s")

    @pl.when(sid == 0)
    def _():
        pltpu.sync_copy(s0_hbm.at[c], acc_sh)

    plsc.subcore_barrier()

    per_w = E // NC // NS

    @pl.loop(0, per_w // CHUNK)
    def _(j):
        base = pl.multiple_of((c * NS + sid) * per_w + j * CHUNK, CHUNK)
        pltpu.sync_copy(dst_hbm.at[pl.ds(base, CHUNK)], idx_v)
        pltpu.sync_copy(msgs_hbm.at[pl.ds(base, CHUNK)], upd_v)
        pltpu.sync_copy(upd_v, acc_sh.at[idx_v], add=True)

    plsc.subcore_barrier()

    @pl.when(sid == 0)
    def _():
        pltpu.sync_copy(acc_sh, out_hbm.at[c])


@functools.partial(
    pl.kernel,
    out_type=jax.ShapeDtypeStruct((4, N, 4 * Q), jnp.float32),
    mesh=_VMESH,
    scratch_types=[
        pltpu.VMEM_SHARED((N, 4 * Q), jnp.float32),
        pltpu.VMEM((CHUNK,), jnp.int32),
        pltpu.VMEM((CHUNK,), jnp.int32),
        pltpu.VMEM((CHUNK, 2 * Q), jnp.float32),
        pltpu.VMEM((CHUNK, 4 * Q), jnp.float32),
        pltpu.VMEM((CHUNK, 4 * Q), jnp.float32),
        pltpu.VMEM((CHUNK, 16), jnp.float32),
        pltpu.SemaphoreType.DMA,
    ],
)
def _sc_scatter_v(vq_hbm, msgv_hbm, unit_hbm, src_hbm, dst_hbm, out_hbm,
                  acc_sh, sidx_v, didx_v, mv_v, vsrc_v, upd_v, unit_v, sem):
    c = lax.axis_index("c")
    sid = lax.axis_index("s")
    per_w = E // NS

    @pl.loop(0, CHUNK)
    def _(i):
        upd_v[i, pl.ds(3 * Q, 16)] = jnp.zeros((16,), jnp.float32)
        upd_v[i, pl.ds(3 * Q + 16, 16)] = jnp.zeros((16,), jnp.float32)

    for p in range(2):
        q = c * 2 + p

        @pl.when(sid == 0)
        def _():
            pltpu.sync_copy(vq_hbm.at[pl.ds(pl.multiple_of(q * N, 8), N)],
                            acc_sh)

        plsc.subcore_barrier()

        @pl.loop(0, per_w // CHUNK)
        def _(j):
            base = pl.multiple_of(sid * per_w + j * CHUNK, CHUNK)
            pltpu.sync_copy(src_hbm.at[pl.ds(base, CHUNK)], sidx_v)
            for k in range(CHUNK // 16):
                sidx_v[pl.ds(16 * k, 16)] = sidx_v[pl.ds(16 * k, 16)] + q * N
            pltpu.sync_copy(dst_hbm.at[pl.ds(base, CHUNK)], didx_v)
            pltpu.sync_copy(msgv_hbm.at[q, pl.ds(base, CHUNK)], mv_v)
            pltpu.sync_copy(unit_hbm.at[pl.ds(base, CHUNK)], unit_v)
            pltpu.async_copy(vq_hbm.at[sidx_v], vsrc_v, sem).wait()

            @pl.loop(0, CHUNK)
            def _(i):
                row = unit_v[i]
                us = [
                    lax.gather(
                        row,
                        jnp.full((16, 1), cc, jnp.int32),
                        lax.GatherDimensionNumbers(
                            offset_dims=(), collapsed_slice_dims=(0,),
                            start_index_map=(0,)),
                        (1,),
                        mode=lax.GatherScatterMode.PROMISE_IN_BOUNDS)
                    for cc in range(3)
                ]
                for g in range(Q // 16):
                    dvv_g = mv_v[i, pl.ds(16 * g, 16)]
                    dvr_g = mv_v[i, pl.ds(Q + 16 * g, 16)]
                    for cc in range(3):
                        off = cc * Q + 16 * g
                        upd_v[i, pl.ds(off, 16)] = (
                            dvv_g * vsrc_v[i, pl.ds(off, 16)] + dvr_g * us[cc])

            pltpu.sync_copy(upd_v, acc_sh.at[didx_v], add=True)

        plsc.subcore_barrier()

        @pl.when(sid == 0)
        def _():
            pltpu.sync_copy(acc_sh, out_hbm.at[q])

        plsc.subcore_barrier()


def kernel(s, v, edge_index, rbf, unit, W1f, b1f, W2f, b2f, W1s, b1s, W2s, b2s):
    src = edge_index[0]
    dst = edge_index[1]

    vt = jnp.transpose(v.reshape(N, 3, 4, Q), (2, 0, 1, 3))
    vq = jnp.pad(vt, ((0, 0), (0, 0), (0, 1), (0, 0))).reshape(4 * N, 4 * Q)
    s0 = jnp.stack([s, jnp.zeros_like(s)])
    unitp = jnp.pad(unit, ((0, 0), (0, 13)))

    s_src = _sc_gather_s(s, src)
    msgs, msgv = _edge_mlp(rbf, s_src, W1f, b1f, W2f, b2f, W1s, b1s, W2s, b2s)

    s_out2 = _sc_scatter_s(s0, msgs, dst)
    v_out4 = _sc_scatter_v(vq, msgv, unitp, src, dst)

    s_out = s_out2[0] + s_out2[1]
    v_out = jnp.transpose(v_out4.reshape(4, N, 4, Q)[:, :, :3, :],
                          (1, 2, 0, 3)).reshape(N, 3, H)
    return (s_out, v_out)


# speedup vs baseline: 17.5317x; 1.5552x over previous
import functools

import jax
import jax.numpy as jnp
from jax import lax
from jax.experimental import pallas as pl
from jax.experimental.pallas import tpu as pltpu
from jax.experimental.pallas import tpu_sc as plsc

N = 10000
E = 320000
H = 128
Q = 32
NR = 20
BE = 2000

NC = 2
NS = 16
CHUNK = 80


def _edge_mlp_body(rbf_ref, ssrc_ref, unitp_ref, w1f_ref, b1f_ref, w2f_ref,
                   b2f_ref, w1s_ref, b1s_ref, w2s_ref, b2s_ref,
                   msgs_ref, msgv_ref):
    h1 = jax.nn.silu(
        jnp.dot(rbf_ref[...], w1f_ref[...], preferred_element_type=jnp.float32)
        + b1f_ref[...])
    filt = jnp.dot(h1, w2f_ref[...], preferred_element_type=jnp.float32) + b2f_ref[...]
    h2 = jax.nn.silu(
        jnp.dot(ssrc_ref[...], w1s_ref[...], preferred_element_type=jnp.float32)
        + b1s_ref[...])
    scal = jnp.dot(h2, w2s_ref[...], preferred_element_type=jnp.float32) + b2s_ref[...]
    msg = filt * scal
    ds = msg[:, :H]
    dvv = msg[:, H:2 * H]
    dvr = msg[:, 2 * H:]
    msgs_ref[...] = ds
    for q in range(4):
        msgv_ref[q] = jnp.concatenate(
            [dvv[:, Q * q:Q * (q + 1)], dvr[:, Q * q:Q * (q + 1)],
             unitp_ref[...]], axis=-1)


def _edge_mlp(rbf, s_src, unitp, W1f, b1f, W2f, b2f, W1s, b1s, W2s, b2s):
    return pl.pallas_call(
        _edge_mlp_body,
        grid=(E // BE,),
        in_specs=[
            pl.BlockSpec((BE, NR), lambda i: (i, 0)),
            pl.BlockSpec((BE, H), lambda i: (i, 0)),
            pl.BlockSpec((BE, 16), lambda i: (i, 0)),
            pl.BlockSpec((NR, H), lambda i: (0, 0)),
            pl.BlockSpec((1, H), lambda i: (0, 0)),
            pl.BlockSpec((H, 3 * H), lambda i: (0, 0)),
            pl.BlockSpec((1, 3 * H), lambda i: (0, 0)),
            pl.BlockSpec((H, H), lambda i: (0, 0)),
            pl.BlockSpec((1, H), lambda i: (0, 0)),
            pl.BlockSpec((H, 3 * H), lambda i: (0, 0)),
            pl.BlockSpec((1, 3 * H), lambda i: (0, 0)),
        ],
        out_specs=[
            pl.BlockSpec((BE, H), lambda i: (i, 0)),
            pl.BlockSpec((4, BE, 2 * Q + 16), lambda i: (0, i, 0)),
        ],
        out_shape=[
            jax.ShapeDtypeStruct((E, H), jnp.float32),
            jax.ShapeDtypeStruct((4, E, 2 * Q + 16), jnp.float32),
        ],
    )(rbf, s_src, unitp, W1f, b1f.reshape(1, H), W2f, b2f.reshape(1, 3 * H),
      W1s, b1s.reshape(1, H), W2s, b2s.reshape(1, 3 * H))


_VMESH = plsc.VectorSubcoreMesh(core_axis_name="c", subcore_axis_name="s",
                                num_cores=NC, num_subcores=NS)


@functools.partial(
    pl.kernel,
    out_type=jax.ShapeDtypeStruct((E, H), jnp.float32),
    mesh=_VMESH,
    scratch_types=[
        pltpu.VMEM((CHUNK,), jnp.int32),
        pltpu.VMEM((CHUNK, H), jnp.float32),
        pltpu.SemaphoreType.DMA,
    ],
)
def _sc_gather_s(s_hbm, src_hbm, out_hbm, idx_v, rows_v, sem):
    wid = lax.axis_index("s") * NC + lax.axis_index("c")
    per_w = E // (NC * NS)

    @pl.loop(0, per_w // CHUNK)
    def _(j):
        base = pl.multiple_of(wid * per_w + j * CHUNK, CHUNK)
        pltpu.sync_copy(src_hbm.at[pl.ds(base, CHUNK)], idx_v)
        pltpu.async_copy(s_hbm.at[idx_v], rows_v, sem).wait()
        pltpu.sync_copy(rows_v, out_hbm.at[pl.ds(base, CHUNK)])


@functools.partial(
    pl.kernel,
    out_type=jax.ShapeDtypeStruct((NC, N, H), jnp.float32),
    mesh=_VMESH,
    scratch_types=[
        pltpu.VMEM_SHARED((N, H), jnp.float32),
        pltpu.VMEM((CHUNK,), jnp.int32),
        pltpu.VMEM((CHUNK, H), jnp.float32),
    ],
)
def _sc_scatter_s(s0_hbm, msgs_hbm, dst_hbm, out_hbm, acc_sh, idx_v, upd_v):
    c = lax.axis_index("c")
    sid = lax.axis_index("s")

    @pl.when(sid == 0)
    def _():
        pltpu.sync_copy(s0_hbm.at[c], acc_sh)

    plsc.subcore_barrier()

    per_w = E // NC // NS

    @pl.loop(0, per_w // CHUNK)
    def _(j):
        base = pl.multiple_of((c * NS + sid) * per_w + j * CHUNK, CHUNK)
        pltpu.sync_copy(dst_hbm.at[pl.ds(base, CHUNK)], idx_v)
        pltpu.sync_copy(msgs_hbm.at[pl.ds(base, CHUNK)], upd_v)
        pltpu.sync_copy(upd_v, acc_sh.at[idx_v], add=True)

    plsc.subcore_barrier()

    @pl.when(sid == 0)
    def _():
        pltpu.sync_copy(acc_sh, out_hbm.at[c])


def _broadcast_lane(row, cc):
    return lax.gather(
        row,
        jnp.full((16, 1), cc, jnp.int32),
        lax.GatherDimensionNumbers(
            offset_dims=(), collapsed_slice_dims=(0,), start_index_map=(0,)),
        (1,),
        mode=lax.GatherScatterMode.PROMISE_IN_BOUNDS)


@functools.partial(
    pl.kernel,
    out_type=jax.ShapeDtypeStruct((4, N, 4 * Q), jnp.float32),
    mesh=_VMESH,
    scratch_types=[
        pltpu.VMEM_SHARED((N, 4 * Q), jnp.float32),
        pltpu.VMEM((2, CHUNK), jnp.int32),
        pltpu.VMEM((2, CHUNK), jnp.int32),
        pltpu.VMEM((CHUNK, 2 * Q + 16), jnp.float32),
        pltpu.VMEM((2, CHUNK, 4 * Q), jnp.float32),
        pltpu.VMEM((CHUNK, 4 * Q), jnp.float32),
        pltpu.SemaphoreType.DMA((2,)),
        pltpu.SemaphoreType.DMA((2,)),
        pltpu.SemaphoreType.DMA,
        pltpu.SemaphoreType.DMA((2,)),
    ],
)
def _sc_scatter_v(vq_hbm, msgv_hbm, srcq_hbm, dst_hbm, out_hbm,
                  acc_sh, sidx_v, didx_v, mv_v, vsrc_v, upd_v,
                  sem_si, sem_in, sem_mv, sem_g):
    c = lax.axis_index("c")
    sid = lax.axis_index("s")
    per_w = E // NS
    n = per_w // CHUNK

    @pl.loop(0, CHUNK)
    def _(i):
        upd_v[i, pl.ds(3 * Q, 16)] = jnp.zeros((16,), jnp.float32)
        upd_v[i, pl.ds(3 * Q + 16, 16)] = jnp.zeros((16,), jnp.float32)

    for p in range(2):
        q = c * 2 + p

        def _base(j):
            return pl.multiple_of(sid * per_w + j * CHUNK, CHUNK)

        def _qbase(j):
            return pl.multiple_of(q * E + sid * per_w + j * CHUNK, CHUNK)

        def _issue_inputs(j, b):
            pltpu.async_copy(srcq_hbm.at[pl.ds(_qbase(j), CHUNK)],
                             sidx_v.at[b], sem_si.at[b])
            pltpu.async_copy(dst_hbm.at[pl.ds(_base(j), CHUNK)],
                             didx_v.at[b], sem_in.at[b])

        def _wait_inputs(j, b):
            pltpu.make_async_copy(dst_hbm.at[pl.ds(_base(j), CHUNK)],
                                  didx_v.at[b], sem_in.at[b]).wait()

        def _issue_mv(j):
            pltpu.async_copy(msgv_hbm.at[q, pl.ds(_base(j), CHUNK)],
                             mv_v, sem_mv)

        def _wait_mv(j):
            pltpu.make_async_copy(msgv_hbm.at[q, pl.ds(_base(j), CHUNK)],
                                  mv_v, sem_mv).wait()

        def _issue_gather(b):
            pltpu.async_copy(vq_hbm.at[sidx_v.at[b]], vsrc_v.at[b],
                             sem_g.at[b])

        def _wait_gather(b):
            pltpu.make_async_copy(vq_hbm.at[sidx_v.at[b]], vsrc_v.at[b],
                                  sem_g.at[b]).wait()

        def _wait_sidx(j, b):
            pltpu.make_async_copy(srcq_hbm.at[pl.ds(_qbase(j), CHUNK)],
                                  sidx_v.at[b], sem_si.at[b]).wait()

        @pl.when(sid == 0)
        def _():
            pltpu.sync_copy(vq_hbm.at[pl.ds(pl.multiple_of(q * N, 8), N)],
                            acc_sh)

        plsc.subcore_barrier()

        _issue_inputs(0, 0)
        _issue_inputs(1, 1)
        _issue_mv(0)
        _wait_sidx(0, 0)
        _issue_gather(0)

        @pl.loop(0, n // 2)
        def _(m):
            for b in range(2):
                j = m * 2 + b
                nb = 1 - b
                _wait_gather(b)
                _wait_inputs(j, b)
                _wait_mv(j)

                @pl.loop(0, CHUNK)
                def _(i):
                    row = mv_v[i, pl.ds(2 * Q, 16)]
                    us = [_broadcast_lane(row, cc) for cc in range(3)]
                    for g in range(Q // 16):
                        dvv_g = mv_v[i, pl.ds(16 * g, 16)]
                        dvr_g = mv_v[i, pl.ds(Q + 16 * g, 16)]
                        for cc in range(3):
                            off = cc * Q + 16 * g
                            upd_v[i, pl.ds(off, 16)] = (
                                dvv_g * vsrc_v[b, i, pl.ds(off, 16)]
                                + dvr_g * us[cc])

                @pl.when(j + 1 < n)
                def _():
                    _issue_mv(j + 1)

                pltpu.sync_copy(upd_v, acc_sh.at[didx_v.at[b]], add=True)

                @pl.when(j + 1 < n)
                def _():
                    _wait_sidx(j + 1, nb)
                    _issue_gather(nb)

                @pl.when(j + 2 < n)
                def _():
                    _issue_inputs(j + 2, b)

        plsc.subcore_barrier()

        @pl.when(sid == 0)
        def _():
            pltpu.sync_copy(acc_sh, out_hbm.at[q])

        plsc.subcore_barrier()


def kernel(s, v, edge_index, rbf, unit, W1f, b1f, W2f, b2f, W1s, b1s, W2s, b2s):
    src = edge_index[0]
    dst = edge_index[1]

    vt = jnp.transpose(v.reshape(N, 3, 4, Q), (2, 0, 1, 3))
    vq = jnp.pad(vt, ((0, 0), (0, 0), (0, 1), (0, 0))).reshape(4 * N, 4 * Q)
    s0 = jnp.stack([s, jnp.zeros_like(s)])
    unitp = jnp.pad(unit, ((0, 0), (0, 13)))
    srcq = (src[None, :] + (N * jnp.arange(4, dtype=jnp.int32))[:, None]
            ).reshape(4 * E)

    s_src = _sc_gather_s(s, src)
    msgs, msgv = _edge_mlp(rbf, s_src, unitp, W1f, b1f, W2f, b2f,
                           W1s, b1s, W2s, b2s)

    s_out2 = _sc_scatter_s(s0, msgs, dst)
    v_out4 = _sc_scatter_v(vq, msgv, srcq, dst)

    s_out = s_out2[0] + s_out2[1]
    v_out = jnp.transpose(v_out4.reshape(4, N, 4, Q)[:, :, :3, :],
                          (1, 2, 0, 3)).reshape(N, 3, H)
    return (s_out, v_out)


# speedup vs baseline: 18.1282x; 1.0340x over previous
import functools

import jax
import jax.numpy as jnp
from jax import lax
from jax.experimental import pallas as pl
from jax.experimental.pallas import tpu as pltpu
from jax.experimental.pallas import tpu_sc as plsc

N = 10000
E = 320000
H = 128
Q = 32
NR = 20
BE = 2000

NC = 2
NS = 16
CHUNK = 80


def _edge_mlp_body(rbf_ref, ssrc_ref, unitp_ref, w1f_ref, b1f_ref, w2f_ref,
                   b2f_ref, w1s_ref, b1s_ref, w2s_ref, b2s_ref,
                   msgs_ref, msgv_ref):
    h1 = jax.nn.silu(
        jnp.dot(rbf_ref[...], w1f_ref[...], preferred_element_type=jnp.float32)
        + b1f_ref[...])
    filt = jnp.dot(h1, w2f_ref[...], preferred_element_type=jnp.float32) + b2f_ref[...]
    h2 = jax.nn.silu(
        jnp.dot(ssrc_ref[...], w1s_ref[...], preferred_element_type=jnp.float32)
        + b1s_ref[...])
    scal = jnp.dot(h2, w2s_ref[...], preferred_element_type=jnp.float32) + b2s_ref[...]
    msg = filt * scal
    ds = msg[:, :H]
    dvv = msg[:, H:2 * H]
    dvr = msg[:, 2 * H:]
    msgs_ref[...] = ds
    for q in range(4):
        msgv_ref[q] = jnp.concatenate(
            [dvv[:, Q * q:Q * (q + 1)], dvr[:, Q * q:Q * (q + 1)],
             unitp_ref[...]], axis=-1)


def _edge_mlp(rbf, s_src, unitp, W1f, b1f, W2f, b2f, W1s, b1s, W2s, b2s):
    return pl.pallas_call(
        _edge_mlp_body,
        grid=(E // BE,),
        in_specs=[
            pl.BlockSpec((BE, NR), lambda i: (i, 0)),
            pl.BlockSpec((BE, H), lambda i: (i, 0)),
            pl.BlockSpec((BE, 16), lambda i: (i, 0)),
            pl.BlockSpec((NR, H), lambda i: (0, 0)),
            pl.BlockSpec((1, H), lambda i: (0, 0)),
            pl.BlockSpec((H, 3 * H), lambda i: (0, 0)),
            pl.BlockSpec((1, 3 * H), lambda i: (0, 0)),
            pl.BlockSpec((H, H), lambda i: (0, 0)),
            pl.BlockSpec((1, H), lambda i: (0, 0)),
            pl.BlockSpec((H, 3 * H), lambda i: (0, 0)),
            pl.BlockSpec((1, 3 * H), lambda i: (0, 0)),
        ],
        out_specs=[
            pl.BlockSpec((BE, H), lambda i: (i, 0)),
            pl.BlockSpec((4, BE, 2 * Q + 16), lambda i: (0, i, 0)),
        ],
        out_shape=[
            jax.ShapeDtypeStruct((E, H), jnp.float32),
            jax.ShapeDtypeStruct((4, E, 2 * Q + 16), jnp.float32),
        ],
    )(rbf, s_src, unitp, W1f, b1f.reshape(1, H), W2f, b2f.reshape(1, 3 * H),
      W1s, b1s.reshape(1, H), W2s, b2s.reshape(1, 3 * H))


_VMESH = plsc.VectorSubcoreMesh(core_axis_name="c", subcore_axis_name="s",
                                num_cores=NC, num_subcores=NS)


@functools.partial(
    pl.kernel,
    out_type=jax.ShapeDtypeStruct((E, H), jnp.float32),
    mesh=_VMESH,
    scratch_types=[
        pltpu.VMEM((CHUNK,), jnp.int32),
        pltpu.VMEM((CHUNK, H), jnp.float32),
        pltpu.SemaphoreType.DMA,
    ],
)
def _sc_gather_s(s_hbm, src_hbm, out_hbm, idx_v, rows_v, sem):
    wid = lax.axis_index("s") * NC + lax.axis_index("c")
    per_w = E // (NC * NS)

    @pl.loop(0, per_w // CHUNK)
    def _(j):
        base = pl.multiple_of(wid * per_w + j * CHUNK, CHUNK)
        pltpu.sync_copy(src_hbm.at[pl.ds(base, CHUNK)], idx_v)
        pltpu.async_copy(s_hbm.at[idx_v], rows_v, sem).wait()
        pltpu.sync_copy(rows_v, out_hbm.at[pl.ds(base, CHUNK)])


@functools.partial(
    pl.kernel,
    out_type=jax.ShapeDtypeStruct((NC, N, H), jnp.float32),
    mesh=_VMESH,
    scratch_types=[
        pltpu.VMEM_SHARED((N, H), jnp.float32),
        pltpu.VMEM((CHUNK,), jnp.int32),
        pltpu.VMEM((CHUNK, H), jnp.float32),
    ],
)
def _sc_scatter_s(s0_hbm, msgs_hbm, dst_hbm, out_hbm, acc_sh, idx_v, upd_v):
    c = lax.axis_index("c")
    sid = lax.axis_index("s")

    @pl.when(sid == 0)
    def _():
        pltpu.sync_copy(s0_hbm.at[c], acc_sh)

    plsc.subcore_barrier()

    per_w = E // NC // NS

    @pl.loop(0, per_w // CHUNK)
    def _(j):
        base = pl.multiple_of((c * NS + sid) * per_w + j * CHUNK, CHUNK)
        pltpu.sync_copy(dst_hbm.at[pl.ds(base, CHUNK)], idx_v)
        pltpu.sync_copy(msgs_hbm.at[pl.ds(base, CHUNK)], upd_v)
        pltpu.sync_copy(upd_v, acc_sh.at[idx_v], add=True)

    plsc.subcore_barrier()

    @pl.when(sid == 0)
    def _():
        pltpu.sync_copy(acc_sh, out_hbm.at[c])


def _broadcast_lane(row, cc):
    return lax.gather(
        row,
        jnp.full((16, 1), cc, jnp.int32),
        lax.GatherDimensionNumbers(
            offset_dims=(), collapsed_slice_dims=(0,), start_index_map=(0,)),
        (1,),
        mode=lax.GatherScatterMode.PROMISE_IN_BOUNDS)


@functools.partial(
    pl.kernel,
    out_type=jax.ShapeDtypeStruct((4, N, 4 * Q), jnp.float32),
    mesh=_VMESH,
    scratch_types=[
        pltpu.VMEM_SHARED((N, 4 * Q), jnp.float32),
        pltpu.VMEM((2, CHUNK), jnp.int32),
        pltpu.VMEM((2, CHUNK), jnp.int32),
        pltpu.VMEM((CHUNK, 2 * Q + 16), jnp.float32),
        pltpu.VMEM((2, CHUNK, 4 * Q), jnp.float32),
        pltpu.SemaphoreType.DMA((2,)),
        pltpu.SemaphoreType.DMA((2,)),
        pltpu.SemaphoreType.DMA,
        pltpu.SemaphoreType.DMA((2,)),
        pltpu.SemaphoreType.DMA((2,)),
    ],
)
def _sc_scatter_v(vq_hbm, msgv_hbm, srcq_hbm, dst_hbm, out_hbm,
                  acc_sh, sidx_v, didx_v, mv_v, upd_v,
                  sem_si, sem_di, sem_mv, sem_g, sem_s):
    c = lax.axis_index("c")
    sid = lax.axis_index("s")
    per_w = E // NS
    n = per_w // CHUNK

    for p in range(2):
        q = c * 2 + p

        def _base(j):
            return pl.multiple_of(sid * per_w + j * CHUNK, CHUNK)

        def _qbase(j):
            return pl.multiple_of(q * E + sid * per_w + j * CHUNK, CHUNK)

        def _issue_sidx(j, b):
            pltpu.async_copy(srcq_hbm.at[pl.ds(_qbase(j), CHUNK)],
                             sidx_v.at[b], sem_si.at[b])

        def _wait_sidx(j, b):
            pltpu.make_async_copy(srcq_hbm.at[pl.ds(_qbase(j), CHUNK)],
                                  sidx_v.at[b], sem_si.at[b]).wait()

        def _issue_didx(j, b):
            pltpu.async_copy(dst_hbm.at[pl.ds(_base(j), CHUNK)],
                             didx_v.at[b], sem_di.at[b])

        def _wait_didx(j, b):
            pltpu.make_async_copy(dst_hbm.at[pl.ds(_base(j), CHUNK)],
                                  didx_v.at[b], sem_di.at[b]).wait()

        def _issue_mv(j):
            pltpu.async_copy(msgv_hbm.at[q, pl.ds(_base(j), CHUNK)],
                             mv_v, sem_mv)

        def _wait_mv(j):
            pltpu.make_async_copy(msgv_hbm.at[q, pl.ds(_base(j), CHUNK)],
                                  mv_v, sem_mv).wait()

        def _issue_gather(b):
            pltpu.async_copy(vq_hbm.at[sidx_v.at[b]], upd_v.at[b],
                             sem_g.at[b])

        def _wait_gather(b):
            pltpu.make_async_copy(vq_hbm.at[sidx_v.at[b]], upd_v.at[b],
                                  sem_g.at[b]).wait()

        def _issue_scatter(b):
            pltpu.async_copy(upd_v.at[b], acc_sh.at[didx_v.at[b]],
                             sem_s.at[b], add=True)

        def _wait_scatter(b):
            pltpu.make_async_copy(upd_v.at[b], acc_sh.at[didx_v.at[b]],
                                  sem_s.at[b]).wait()

        @pl.when(sid == 0)
        def _():
            pltpu.sync_copy(vq_hbm.at[pl.ds(pl.multiple_of(q * N, 8), N)],
                            acc_sh)

        plsc.subcore_barrier()

        _issue_sidx(0, 0)
        _issue_sidx(1, 1)
        _issue_didx(0, 0)
        _issue_mv(0)
        _wait_sidx(0, 0)
        _issue_gather(0)

        @pl.loop(0, n // 2)
        def _(m):
            for b in range(2):
                j = m * 2 + b
                nb = 1 - b
                _wait_gather(b)
                _wait_didx(j, b)
                _wait_mv(j)

                @pl.loop(0, CHUNK)
                def _(i):
                    row = mv_v[i, pl.ds(2 * Q, 16)]
                    us = [_broadcast_lane(row, cc) for cc in range(3)]
                    for g in range(Q // 16):
                        dvv_g = mv_v[i, pl.ds(16 * g, 16)]
                        dvr_g = mv_v[i, pl.ds(Q + 16 * g, 16)]
                        for cc in range(3):
                            off = cc * Q + 16 * g
                            upd_v[b, i, pl.ds(off, 16)] = (
                                dvv_g * upd_v[b, i, pl.ds(off, 16)]
                                + dvr_g * us[cc])

                @pl.when(j + 1 < n)
                def _():
                    _issue_mv(j + 1)

                @pl.when(j >= 1)
                def _():
                    _wait_scatter(nb)

                _issue_scatter(b)

                @pl.when(j + 1 < n)
                def _():
                    _wait_sidx(j + 1, nb)
                    _issue_gather(nb)
                    _issue_didx(j + 1, nb)

                @pl.when(j + 2 < n)
                def _():
                    _issue_sidx(j + 2, b)

        _wait_scatter(1)
        plsc.subcore_barrier()

        @pl.when(sid == 0)
        def _():
            pltpu.sync_copy(acc_sh, out_hbm.at[q])

        plsc.subcore_barrier()


def kernel(s, v, edge_index, rbf, unit, W1f, b1f, W2f, b2f, W1s, b1s, W2s, b2s):
    src = edge_index[0]
    dst = edge_index[1]

    vt = jnp.transpose(v.reshape(N, 3, 4, Q), (2, 0, 1, 3))
    vq = jnp.pad(vt, ((0, 0), (0, 0), (0, 1), (0, 0))).reshape(4 * N, 4 * Q)
    s0 = jnp.stack([s, jnp.zeros_like(s)])
    unitp = jnp.pad(unit, ((0, 0), (0, 13)))
    srcq = (src[None, :] + (N * jnp.arange(4, dtype=jnp.int32))[:, None]
            ).reshape(4 * E)

    s_src = _sc_gather_s(s, src)
    msgs, msgv = _edge_mlp(rbf, s_src, unitp, W1f, b1f, W2f, b2f,
                           W1s, b1s, W2s, b2s)

    s_out2 = _sc_scatter_s(s0, msgs, dst)
    v_out4 = _sc_scatter_v(vq, msgv, srcq, dst)

    s_out = s_out2[0] + s_out2[1]
    v_out = jnp.transpose(v_out4.reshape(4, N, 4, Q)[:, :, :3, :],
                          (1, 2, 0, 3)).reshape(N, 3, H)
    return (s_out, v_out)


# speedup vs baseline: 18.9191x; 1.0436x over previous
import functools

import jax
import jax.numpy as jnp
from jax import lax
from jax.experimental import pallas as pl
from jax.experimental.pallas import tpu as pltpu
from jax.experimental.pallas import tpu_sc as plsc

N = 10000
E = 320000
H = 128
Q = 32
NR = 20
BE = 2000

NC = 2
NS = 16
CHUNK = 80


def _edge_mlp_body(rbf_ref, ssrc_ref, unitp_ref, w1f_ref, b1f_ref, w2f_ref,
                   b2f_ref, w1s_ref, b1s_ref, w2s_ref, b2s_ref,
                   msgs_ref, msgv_ref):
    bf = jnp.bfloat16
    f32 = jnp.float32
    h1 = jax.nn.silu(
        jnp.dot(rbf_ref[...].astype(bf), w1f_ref[...].astype(bf),
                preferred_element_type=f32) + b1f_ref[...])
    filt = jnp.dot(h1.astype(bf), w2f_ref[...].astype(bf),
                   preferred_element_type=f32) + b2f_ref[...]
    h2 = jax.nn.silu(
        jnp.dot(ssrc_ref[...].astype(bf), w1s_ref[...].astype(bf),
                preferred_element_type=f32) + b1s_ref[...])
    scal = jnp.dot(h2.astype(bf), w2s_ref[...].astype(bf),
                   preferred_element_type=f32) + b2s_ref[...]
    msg = filt * scal
    ds = msg[:, :H]
    dvv = msg[:, H:2 * H]
    dvr = msg[:, 2 * H:]
    msgs_ref[...] = ds
    ubc = [jnp.broadcast_to(unitp_ref[...][:, cc:cc + 1], (BE, 16))
           for cc in range(3)]
    for q in range(4):
        msgv_ref[q] = jnp.concatenate(
            [dvv[:, Q * q:Q * (q + 1)], dvr[:, Q * q:Q * (q + 1)],
             ubc[0], ubc[1], ubc[2]], axis=-1)


def _edge_mlp(rbf, s_src, unitp, W1f, b1f, W2f, b2f, W1s, b1s, W2s, b2s):
    return pl.pallas_call(
        _edge_mlp_body,
        grid=(E // BE,),
        in_specs=[
            pl.BlockSpec((BE, NR), lambda i: (i, 0)),
            pl.BlockSpec((BE, H), lambda i: (i, 0)),
            pl.BlockSpec((BE, 16), lambda i: (i, 0)),
            pl.BlockSpec((NR, H), lambda i: (0, 0)),
            pl.BlockSpec((1, H), lambda i: (0, 0)),
            pl.BlockSpec((H, 3 * H), lambda i: (0, 0)),
            pl.BlockSpec((1, 3 * H), lambda i: (0, 0)),
            pl.BlockSpec((H, H), lambda i: (0, 0)),
            pl.BlockSpec((1, H), lambda i: (0, 0)),
            pl.BlockSpec((H, 3 * H), lambda i: (0, 0)),
            pl.BlockSpec((1, 3 * H), lambda i: (0, 0)),
        ],
        out_specs=[
            pl.BlockSpec((BE, H), lambda i: (i, 0)),
            pl.BlockSpec((4, BE, 2 * Q + 48), lambda i: (0, i, 0)),
        ],
        out_shape=[
            jax.ShapeDtypeStruct((E, H), jnp.float32),
            jax.ShapeDtypeStruct((4, E, 2 * Q + 48), jnp.float32),
        ],
    )(rbf, s_src, unitp, W1f, b1f.reshape(1, H), W2f, b2f.reshape(1, 3 * H),
      W1s, b1s.reshape(1, H), W2s, b2s.reshape(1, 3 * H))


_VMESH = plsc.VectorSubcoreMesh(core_axis_name="c", subcore_axis_name="s",
                                num_cores=NC, num_subcores=NS)


@functools.partial(
    pl.kernel,
    out_type=jax.ShapeDtypeStruct((E, H), jnp.float32),
    mesh=_VMESH,
    scratch_types=[
        pltpu.VMEM((2, CHUNK), jnp.int32),
        pltpu.VMEM((2, CHUNK, H), jnp.float32),
        pltpu.SemaphoreType.DMA((2,)),
        pltpu.SemaphoreType.DMA((2,)),
        pltpu.SemaphoreType.DMA((2,)),
    ],
)
def _sc_gather_s(s_hbm, src_hbm, out_hbm, idx_v, rows_v, sem_i, sem_g, sem_w):
    wid = lax.axis_index("s") * NC + lax.axis_index("c")
    per_w = E // (NC * NS)
    n = per_w // CHUNK

    def _base(j):
        return pl.multiple_of(wid * per_w + j * CHUNK, CHUNK)

    def _issue_idx(j, b):
        pltpu.async_copy(src_hbm.at[pl.ds(_base(j), CHUNK)], idx_v.at[b],
                         sem_i.at[b])

    def _wait_idx(j, b):
        pltpu.make_async_copy(src_hbm.at[pl.ds(_base(j), CHUNK)],
                              idx_v.at[b], sem_i.at[b]).wait()

    def _issue_gather(b):
        pltpu.async_copy(s_hbm.at[idx_v.at[b]], rows_v.at[b], sem_g.at[b])

    def _wait_gather(b):
        pltpu.make_async_copy(s_hbm.at[idx_v.at[b]], rows_v.at[b],
                              sem_g.at[b]).wait()

    def _issue_wb(j, b):
        pltpu.async_copy(rows_v.at[b], out_hbm.at[pl.ds(_base(j), CHUNK)],
                         sem_w.at[b])

    def _wait_wb(j, b):
        pltpu.make_async_copy(rows_v.at[b],
                              out_hbm.at[pl.ds(_base(j), CHUNK)],
                              sem_w.at[b]).wait()

    _issue_idx(0, 0)
    _issue_idx(1, 1)
    _wait_idx(0, 0)
    _issue_gather(0)

    @pl.loop(0, (n + 1) // 2)
    def _(m):
        for b in range(2):
            j = m * 2 + b
            nb = 1 - b

            @pl.when(j < n)
            def _():
                _wait_gather(b)
                _issue_wb(j, b)

                @pl.when(j + 1 < n)
                def _():
                    _wait_idx(j + 1, nb)

                    @pl.when(j >= 1)
                    def _():
                        _wait_wb(j - 1, nb)

                    _issue_gather(nb)

                @pl.when(j + 2 < n)
                def _():
                    _issue_idx(j + 2, b)

    _wait_wb(n - 2, (n - 2) % 2)
    _wait_wb(n - 1, (n - 1) % 2)


@functools.partial(
    pl.kernel,
    out_type=jax.ShapeDtypeStruct((NC, N, H), jnp.float32),
    mesh=_VMESH,
    scratch_types=[
        pltpu.VMEM_SHARED((N, H), jnp.float32),
        pltpu.VMEM((2, CHUNK), jnp.int32),
        pltpu.VMEM((2, CHUNK, H), jnp.float32),
        pltpu.SemaphoreType.DMA((2,)),
        pltpu.SemaphoreType.DMA((2,)),
    ],
)
def _sc_scatter_s(s0_hbm, msgs_hbm, dst_hbm, out_hbm, acc_sh, idx_v, upd_v,
                  sem_i, sem_u):
    c = lax.axis_index("c")
    sid = lax.axis_index("s")
    per_w = E // NC // NS
    n = per_w // CHUNK

    def _base(j):
        return pl.multiple_of((c * NS + sid) * per_w + j * CHUNK, CHUNK)

    def _issue(j, b):
        pltpu.async_copy(dst_hbm.at[pl.ds(_base(j), CHUNK)], idx_v.at[b],
                         sem_i.at[b])
        pltpu.async_copy(msgs_hbm.at[pl.ds(_base(j), CHUNK)], upd_v.at[b],
                         sem_u.at[b])

    def _wait(j, b):
        pltpu.make_async_copy(dst_hbm.at[pl.ds(_base(j), CHUNK)],
                              idx_v.at[b], sem_i.at[b]).wait()
        pltpu.make_async_copy(msgs_hbm.at[pl.ds(_base(j), CHUNK)],
                              upd_v.at[b], sem_u.at[b]).wait()

    @pl.when(sid == 0)
    def _():
        pltpu.sync_copy(s0_hbm.at[c], acc_sh)

    plsc.subcore_barrier()

    _issue(0, 0)
    _issue(1, 1)

    @pl.loop(0, (n + 1) // 2)
    def _(m):
        for b in range(2):
            j = m * 2 + b

            @pl.when(j < n)
            def _():
                _wait(j, b)
                pltpu.sync_copy(upd_v.at[b], acc_sh.at[idx_v.at[b]], add=True)

                @pl.when(j + 2 < n)
                def _():
                    _issue(j + 2, b)

    plsc.subcore_barrier()

    @pl.when(sid == 0)
    def _():
        pltpu.sync_copy(acc_sh, out_hbm.at[c])


def _broadcast_lane(row, cc):
    return lax.gather(
        row,
        jnp.full((16, 1), cc, jnp.int32),
        lax.GatherDimensionNumbers(
            offset_dims=(), collapsed_slice_dims=(0,), start_index_map=(0,)),
        (1,),
        mode=lax.GatherScatterMode.PROMISE_IN_BOUNDS)


@functools.partial(
    pl.kernel,
    out_type=jax.ShapeDtypeStruct((4, N, 4 * Q), jnp.float32),
    mesh=_VMESH,
    scratch_types=[
        pltpu.VMEM_SHARED((N, 4 * Q), jnp.float32),
        pltpu.VMEM((2, CHUNK), jnp.int32),
        pltpu.VMEM((2, CHUNK), jnp.int32),
        pltpu.VMEM((CHUNK, 2 * Q + 48), jnp.float32),
        pltpu.VMEM((2, CHUNK, 4 * Q), jnp.float32),
        pltpu.SemaphoreType.DMA((2,)),
        pltpu.SemaphoreType.DMA((2,)),
        pltpu.SemaphoreType.DMA,
        pltpu.SemaphoreType.DMA((2,)),
        pltpu.SemaphoreType.DMA((2,)),
    ],
)
def _sc_scatter_v(vq_hbm, msgv_hbm, srcq_hbm, dst_hbm, out_hbm,
                  acc_sh, sidx_v, didx_v, mv_v, upd_v,
                  sem_si, sem_di, sem_mv, sem_g, sem_s):
    c = lax.axis_index("c")
    sid = lax.axis_index("s")
    per_w = E // NS
    n = per_w // CHUNK

    for p in range(2):
        q = c * 2 + p

        def _base(j):
            return pl.multiple_of(sid * per_w + j * CHUNK, CHUNK)

        def _qbase(j):
            return pl.multiple_of(q * E + sid * per_w + j * CHUNK, CHUNK)

        def _issue_sidx(j, b):
            pltpu.async_copy(srcq_hbm.at[pl.ds(_qbase(j), CHUNK)],
                             sidx_v.at[b], sem_si.at[b])

        def _wait_sidx(j, b):
            pltpu.make_async_copy(srcq_hbm.at[pl.ds(_qbase(j), CHUNK)],
                                  sidx_v.at[b], sem_si.at[b]).wait()

        def _issue_didx(j, b):
            pltpu.async_copy(dst_hbm.at[pl.ds(_base(j), CHUNK)],
                             didx_v.at[b], sem_di.at[b])

        def _wait_didx(j, b):
            pltpu.make_async_copy(dst_hbm.at[pl.ds(_base(j), CHUNK)],
                                  didx_v.at[b], sem_di.at[b]).wait()

        def _issue_mv(j):
            pltpu.async_copy(msgv_hbm.at[q, pl.ds(_base(j), CHUNK)],
                             mv_v, sem_mv)

        def _wait_mv(j):
            pltpu.make_async_copy(msgv_hbm.at[q, pl.ds(_base(j), CHUNK)],
                                  mv_v, sem_mv).wait()

        def _issue_gather(b):
            pltpu.async_copy(vq_hbm.at[sidx_v.at[b]], upd_v.at[b],
                             sem_g.at[b])

        def _wait_gather(b):
            pltpu.make_async_copy(vq_hbm.at[sidx_v.at[b]], upd_v.at[b],
                                  sem_g.at[b]).wait()

        def _issue_scatter(b):
            pltpu.async_copy(upd_v.at[b], acc_sh.at[didx_v.at[b]],
                             sem_s.at[b], add=True)

        def _wait_scatter(b):
            pltpu.make_async_copy(upd_v.at[b], acc_sh.at[didx_v.at[b]],
                                  sem_s.at[b]).wait()

        @pl.when(sid == 0)
        def _():
            pltpu.sync_copy(vq_hbm.at[pl.ds(pl.multiple_of(q * N, 8), N)],
                            acc_sh)

        plsc.subcore_barrier()

        _issue_sidx(0, 0)
        _issue_sidx(1, 1)
        _issue_didx(0, 0)
        _issue_mv(0)
        _wait_sidx(0, 0)
        _issue_gather(0)

        @pl.loop(0, n // 2)
        def _(m):
            for b in range(2):
                j = m * 2 + b
                nb = 1 - b
                _wait_gather(b)
                _wait_didx(j, b)
                _wait_mv(j)

                @pl.loop(0, CHUNK)
                def _(i):
                    us = [mv_v[i, pl.ds(2 * Q + 16 * cc, 16)]
                          for cc in range(3)]
                    for g in range(Q // 16):
                        dvv_g = mv_v[i, pl.ds(16 * g, 16)]
                        dvr_g = mv_v[i, pl.ds(Q + 16 * g, 16)]
                        for cc in range(3):
                            off = cc * Q + 16 * g
                            upd_v[b, i, pl.ds(off, 16)] = (
                                dvv_g * upd_v[b, i, pl.ds(off, 16)]
                                + dvr_g * us[cc])

                @pl.when(j + 1 < n)
                def _():
                    _issue_mv(j + 1)

                @pl.when(j >= 1)
                def _():
                    _wait_scatter(nb)

                _issue_scatter(b)

                @pl.when(j + 1 < n)
                def _():
                    _wait_sidx(j + 1, nb)
                    _issue_gather(nb)
                    _issue_didx(j + 1, nb)

                @pl.when(j + 2 < n)
                def _():
                    _issue_sidx(j + 2, b)

        _wait_scatter(1)
        plsc.subcore_barrier()

        @pl.when(sid == 0)
        def _():
            pltpu.sync_copy(acc_sh, out_hbm.at[q])

        plsc.subcore_barrier()


def kernel(s, v, edge_index, rbf, unit, W1f, b1f, W2f, b2f, W1s, b1s, W2s, b2s):
    src = edge_index[0]
    dst = edge_index[1]

    vt = jnp.transpose(v.reshape(N, 3, 4, Q), (2, 0, 1, 3))
    vq = jnp.pad(vt, ((0, 0), (0, 0), (0, 1), (0, 0))).reshape(4 * N, 4 * Q)
    s0 = jnp.stack([s, jnp.zeros_like(s)])
    unitp = jnp.pad(unit, ((0, 0), (0, 13)))
    srcq = (src[None, :] + (N * jnp.arange(4, dtype=jnp.int32))[:, None]
            ).reshape(4 * E)

    s_src = _sc_gather_s(s, src)
    msgs, msgv = _edge_mlp(rbf, s_src, unitp, W1f, b1f, W2f, b2f,
                           W1s, b1s, W2s, b2s)

    s_out2 = _sc_scatter_s(s0, msgs, dst)
    v_out4 = _sc_scatter_v(vq, msgv, srcq, dst)

    s_out = s_out2[0] + s_out2[1]
    v_out = jnp.transpose(v_out4.reshape(4, N, 4, Q)[:, :, :3, :],
                          (1, 2, 0, 3)).reshape(N, 3, H)
    return (s_out, v_out)
